# baseline (speedup 1.0000x reference)
import functools

import numpy as np
import jax
import jax.numpy as jnp
from jax import lax
from jax.experimental import pallas as pl
from jax.experimental.pallas import tpu as pltpu
from jax.experimental.pallas import tpu_sc as plsc

N = 50000
G = 128
CH = 16
NCLASS = 21
SENT = np.int32(1 << 30)
OFFS = np.array([(dx, dy, dz) for dx in (-1, 0, 1) for dy in (-1, 0, 1)
                 for dz in (-1, 0, 1)], np.int32)

NUM_CORES = 2
NUM_SUBCORES = 16
NW = NUM_CORES * NUM_SUBCORES


def _chunk(D):
    return 2048 if D <= 16 else 1024


def _pad_len(n, D):
    step = 2 * _chunk(D)
    per = -(-n // NW)
    per = -(-per // step) * step
    return per * NW


@functools.lru_cache(None)
def _sc_gather_fn(B, D, T, staged):
    C = _chunk(D)
    bpw = B // NW
    n_chunks = bpw // C
    nc2 = n_chunks // 2
    mesh = plsc.VectorSubcoreMesh(core_axis_name="c", subcore_axis_name="s")

    @functools.partial(
        pl.kernel,
        out_type=jax.ShapeDtypeStruct((B, D), jnp.float32),
        scratch_types=[
            pltpu.VMEM((2, C), jnp.int32),
            pltpu.VMEM((2, C, D), jnp.float32),
            (pltpu.VMEM_SHARED((T, D), jnp.float32) if staged else None),
            pltpu.SemaphoreType.DMA,
            pltpu.SemaphoreType.DMA,
        ],
        mesh=mesh,
        compiler_params=pltpu.CompilerParams(use_tc_tiling_on_sc=False),
    )
    def k(table_hbm, idx_hbm, out_hbm, idx_v, rows_v, tbl_s, sg0, sg1):
        sid = lax.axis_index("s")
        wid = sid * NUM_CORES + lax.axis_index("c")
        wbase = pl.multiple_of(wid * bpw, bpw)
        sg = (sg0, sg1)

        if staged:
            @pl.when(sid == 0)
            def _():
                pltpu.sync_copy(table_hbm, tbl_s)

            plsc.subcore_barrier()
            src = tbl_s
        else:
            src = table_hbm

        def load_idx(c, b):
            pltpu.sync_copy(
                idx_hbm.at[pl.ds(pl.multiple_of(wbase + c * C, C), C)],
                idx_v.at[b])

        def fire(b):
            pltpu.async_copy(src.at[idx_v.at[b]], rows_v.at[b], sg[b])

        def wait(b):
            pltpu.make_async_copy(src.at[idx_v.at[b]], rows_v.at[b],
                                  sg[b]).wait()

        def store(c, b):
            pltpu.sync_copy(
                rows_v.at[b],
                out_hbm.at[pl.ds(pl.multiple_of(wbase + c * C, C), C)])

        load_idx(0, 0)
        fire(0)

        def body(i, carry):
            c0 = 2 * i
            load_idx(c0 + 1, 1)
            fire(1)
            wait(0)
            store(c0, 0)

            @pl.when(i < nc2 - 1)
            def _():
                load_idx(c0 + 2, 0)
                fire(0)

            wait(1)
            store(c0 + 1, 1)
            return carry

        lax.fori_loop(0, nc2, body, 0)

    return k


def _sc_gather(table, idx_flat, D):
    B0 = idx_flat.shape[0]
    B = _pad_len(B0, D)
    if B != B0:
        idx_flat = jnp.concatenate(
            [idx_flat, jnp.zeros((B - B0,), jnp.int32)])
    T = table.shape[0]
    staged = T * D <= 900_000
    return _sc_gather_fn(B, D, T, staged)(table, idx_flat)


BLK = 2000
NBLK = N // BLK


@functools.lru_cache(None)
def _conv_fn(B):
    def body(g_ref, w_ref, b_ref, o_ref):
        i = pl.program_id(1)

        @pl.when(i == 0)
        def _():
            o_ref[...] = jnp.zeros_like(o_ref)

        o_ref[...] += jnp.dot(g_ref[...], w_ref[0],
                              preferred_element_type=jnp.float32)

        @pl.when(i == 26)
        def _():
            o_ref[...] = jnp.maximum(o_ref[...] + b_ref[...], 0.0)

    return pl.pallas_call(
        body,
        grid=(NBLK, 27),
        in_specs=[
            pl.BlockSpec((BLK, CH), lambda j, i: (i * NBLK + j, 0)),
            pl.BlockSpec((1, CH, CH), lambda j, i: (i, 0, 0)),
            pl.BlockSpec((1, CH), lambda j, i: (0, 0)),
        ],
        out_specs=pl.BlockSpec((BLK, CH), lambda j, i: (j, 0)),
        out_shape=jax.ShapeDtypeStruct((N, CH), jnp.float32),
    )


@functools.lru_cache(None)
def _conv2_fn(B):
    def body(g1_ref, g2_ref, w1_ref, w2_ref, b_ref, o_ref):
        i = pl.program_id(1)

        @pl.when(i == 0)
        def _():
            o_ref[...] = jnp.zeros_like(o_ref)

        o_ref[...] += (
            jnp.dot(g1_ref[...], w1_ref[0],
                    preferred_element_type=jnp.float32)
            + jnp.dot(g2_ref[...], w2_ref[0],
                      preferred_element_type=jnp.float32))

        @pl.when(i == 26)
        def _():
            o_ref[...] = jnp.maximum(o_ref[...] + b_ref[...], 0.0)

    gspec = pl.BlockSpec((BLK, CH), lambda j, i: (i * NBLK + j, 0))
    wspec = pl.BlockSpec((1, CH, CH), lambda j, i: (i, 0, 0))
    return pl.pallas_call(
        body,
        grid=(NBLK, 27),
        in_specs=[gspec, gspec, wspec, wspec,
                  pl.BlockSpec((1, CH), lambda j, i: (0, 0))],
        out_specs=pl.BlockSpec((BLK, CH), lambda j, i: (j, 0)),
        out_shape=jax.ShapeDtypeStruct((N, CH), jnp.float32),
    )


@functools.lru_cache(None)
def _poolmax_fn(B):
    def body(g_ref, o_ref):
        t = pl.program_id(1)

        @pl.when(t == 0)
        def _():
            o_ref[...] = g_ref[...]

        @pl.when(t > 0)
        def _():
            o_ref[...] = jnp.maximum(o_ref[...], g_ref[...])

    return pl.pallas_call(
        body,
        grid=(NBLK, 8),
        in_specs=[pl.BlockSpec((BLK, CH), lambda j, t: (t * NBLK + j, 0))],
        out_specs=pl.BlockSpec((BLK, CH), lambda j, t: (j, 0)),
        out_shape=jax.ShapeDtypeStruct((N, CH), jnp.float32),
    )


@functools.lru_cache(None)
def _head_fn():
    def body(x_ref, w_ref, b_ref, o_ref):
        o_ref[...] = jnp.dot(x_ref[...], w_ref[...],
                             preferred_element_type=jnp.float32) + b_ref[...]

    return pl.pallas_call(
        body,
        grid=(NBLK,),
        in_specs=[
            pl.BlockSpec((BLK, CH), lambda j: (j, 0)),
            pl.BlockSpec((CH, NCLASS), lambda j: (0, 0)),
            pl.BlockSpec((1, NCLASS), lambda j: (0, 0)),
        ],
        out_specs=pl.BlockSpec((BLK, NCLASS), lambda j: (j, 0)),
        out_shape=jax.ShapeDtypeStruct((N, NCLASS), jnp.float32),
    )


def _lin(c, g):
    return (c[..., 0] * g + c[..., 1]) * g + c[..., 2]


def _neighbor_idx(coords, keys, valid, g):
    g3 = g * g * g
    ar = jnp.arange(N, dtype=jnp.int32)
    x, y, z = coords[:, 0], coords[:, 1], coords[:, 2]
    tbl = jnp.full((g3, 32), np.int32(-1))
    for i, (dx, dy, dz) in enumerate(OFFS):
        ok = (valid & (x - dx >= 0) & (x - dx < g) & (y - dy >= 0)
              & (y - dy < g) & (z - dz >= 0) & (z - dz < g))
        src = jnp.where(ok, keys - (dx * g * g + dy * g + dz), g3)
        tbl = tbl.at[src, i].set(ar, mode="drop")
    q = jnp.where(valid, keys, 0)
    gath = _sc_gather(lax.bitcast_convert_type(tbl, jnp.float32), q, 32)
    pos27 = lax.bitcast_convert_type(gath[:N], jnp.int32)[:, :27]
    gidx = jnp.where(pos27 >= 0, pos27, N).astype(jnp.int32)
    return gidx.T.reshape(-1)


def _pool_build(coords, keys, valid, g):
    gc = g // 2
    gc3 = gc * gc * gc
    ar = jnp.arange(N, dtype=jnp.int32)
    ck = jnp.where(valid, _lin(coords // 2, gc), gc3).astype(jnp.int32)
    occ = jnp.zeros((gc3,), jnp.int32).at[ck].set(1, mode="drop")
    csum = jnp.cumsum(occ)
    M = csum[-1]
    slot = (csum - occ).astype(jnp.int32)
    cells = jnp.zeros((N,), jnp.int32).at[
        jnp.where(occ == 1, slot, N)].set(
            jnp.arange(gc3, dtype=jnp.int32), mode="drop")
    pos_u = jnp.minimum(slot[jnp.clip(ck, 0, gc3 - 1)], N - 1)
    sub = (((coords[:, 0] & 1) << 2) | ((coords[:, 1] & 1) << 1)
           | (coords[:, 2] & 1))
    sub = jnp.where(valid, sub, 8)
    cidx = jnp.full((8, N), N, jnp.int32).at[sub, pos_u].set(ar, mode="drop")
    new_valid = ar < M
    new_keys = jnp.where(new_valid, cells, SENT)
    uk = jnp.where(new_valid, cells, 0)
    new_coords = jnp.stack(
        [uk // (gc * gc), (uk // gc) % gc, uk % gc], axis=1).astype(jnp.int32)
    return cidx.reshape(-1), pos_u, new_coords, new_keys, new_valid


def _ext(feat):
    return jnp.concatenate([feat, jnp.zeros((1, feat.shape[1]), feat.dtype)])


def _conv(feat, gidx, W, b):
    g = _sc_gather(_ext(feat), gidx, CH)
    return _conv_fn(g.shape[0])(g, W, b.reshape(1, CH))


def _conv2(up, skip, gidx, W, b):
    g1 = _sc_gather(_ext(up), gidx, CH)
    g2 = _sc_gather(_ext(skip), gidx, CH)
    return _conv2_fn(g1.shape[0])(
        g1, g2, W[:, :CH, :], W[:, CH:, :], b.reshape(1, CH))


def kernel(voxel_features, voxel_xyz_indices, num_valid_voxels,
           W_b0, W_b1, W_d00, W_d01, W_d10, W_d11, W_d20, W_d21,
           W_e00, W_e01, W_e10, W_e11, W_e20, W_e21, W_h,
           b_b0, b_b1, b_d00, b_d01, b_d10, b_d11, b_d20, b_d21,
           b_e00, b_e01, b_e10, b_e11, b_e20, b_e21, b_h):
    P = dict(W_b0=W_b0, W_b1=W_b1, W_d00=W_d00, W_d01=W_d01, W_d10=W_d10,
             W_d11=W_d11, W_d20=W_d20, W_d21=W_d21, W_e00=W_e00,
             W_e01=W_e01, W_e10=W_e10, W_e11=W_e11, W_e20=W_e20,
             W_e21=W_e21, b_b0=b_b0, b_b1=b_b1, b_d00=b_d00, b_d01=b_d01,
             b_d10=b_d10, b_d11=b_d11, b_d20=b_d20, b_d21=b_d21,
             b_e00=b_e00, b_e01=b_e01, b_e10=b_e10, b_e11=b_e11,
             b_e20=b_e20, b_e21=b_e21)
    feat = voxel_features[0]
    coords = voxel_xyz_indices[0].astype(jnp.int32)

    g = G
    keys = _lin(coords, g).astype(jnp.int32)
    valid = jnp.ones((N,), bool)

    skips = []
    for lvl, blk in enumerate((('e00', 'e01'), ('e10', 'e11'),
                               ('e20', 'e21'))):
        gidx = _neighbor_idx(coords, keys, valid, g)
        for nm in blk:
            feat = _conv(feat, gidx, P['W_' + nm], P['b_' + nm])
        cidx, pos_u, nco, nke, nva = _pool_build(coords, keys, valid, g)
        skips.append((feat, gidx, pos_u))
        gch = _sc_gather(_ext(feat), cidx, CH)
        feat = _poolmax_fn(gch.shape[0])(gch)
        coords, keys, valid, g = nco, nke, nva, g // 2

    gidx = _neighbor_idx(coords, keys, valid, g)
    for nm in ('b0', 'b1'):
        feat = _conv(feat, gidx, P['W_' + nm], P['b_' + nm])

    for blk, (sf, sgidx, pos_u) in zip(
            (('d00', 'd01'), ('d10', 'd11'), ('d20', 'd21')),
            reversed(skips)):
        up = _sc_gather(feat, pos_u, CH)[:N]
        feat = _conv2(up, sf, sgidx, P['W_' + blk[0]], P['b_' + blk[0]])
        feat = _conv(feat, sgidx, P['W_' + blk[1]], P['b_' + blk[1]])

    return _head_fn()(feat, W_h, b_h.reshape(1, NCLASS))


# speedup vs baseline: 2.0517x; 2.0517x over previous
import functools

import numpy as np
import jax
import jax.numpy as jnp
from jax import lax
from jax.experimental import pallas as pl
from jax.experimental.pallas import tpu as pltpu
from jax.experimental.pallas import tpu_sc as plsc

N = 50000
G = 128
CH = 16
NCLASS = 21
SENT = np.int32(1 << 30)
OFFS = np.array([(dx, dy, dz) for dx in (-1, 0, 1) for dy in (-1, 0, 1)
                 for dz in (-1, 0, 1)], np.int32)

NUM_CORES = 2
NUM_SUBCORES = 16
NW = NUM_CORES * NUM_SUBCORES


def _chunk(D):
    return 2048 if D <= 16 else 1024


def _pad_len(n, D):
    step = 2 * _chunk(D)
    per = -(-n // NW)
    per = -(-per // step) * step
    return per * NW


@functools.lru_cache(None)
def _sc_gather_fn(B, D, T, staged):
    C = _chunk(D)
    bpw = B // NW
    n_chunks = bpw // C
    nc2 = n_chunks // 2
    mesh = plsc.VectorSubcoreMesh(core_axis_name="c", subcore_axis_name="s")

    @functools.partial(
        pl.kernel,
        out_type=jax.ShapeDtypeStruct((B, D), jnp.float32),
        scratch_types=[
            pltpu.VMEM((2, C), jnp.int32),
            pltpu.VMEM((2, C, D), jnp.float32),
            (pltpu.VMEM_SHARED((T, D), jnp.float32) if staged else None),
            pltpu.SemaphoreType.DMA,
            pltpu.SemaphoreType.DMA,
        ],
        mesh=mesh,
        compiler_params=pltpu.CompilerParams(use_tc_tiling_on_sc=False),
    )
    def k(table_hbm, idx_hbm, out_hbm, idx_v, rows_v, tbl_s, sg0, sg1):
        sid = lax.axis_index("s")
        wid = sid * NUM_CORES + lax.axis_index("c")
        wbase = pl.multiple_of(wid * bpw, bpw)
        sg = (sg0, sg1)

        if staged:
            @pl.when(sid == 0)
            def _():
                pltpu.sync_copy(table_hbm, tbl_s)

            plsc.subcore_barrier()
            src = tbl_s
        else:
            src = table_hbm

        def load_idx(c, b):
            pltpu.sync_copy(
                idx_hbm.at[pl.ds(pl.multiple_of(wbase + c * C, C), C)],
                idx_v.at[b])

        def fire(b):
            pltpu.async_copy(src.at[idx_v.at[b]], rows_v.at[b], sg[b])

        def wait(b):
            pltpu.make_async_copy(src.at[idx_v.at[b]], rows_v.at[b],
                                  sg[b]).wait()

        def store(c, b):
            pltpu.sync_copy(
                rows_v.at[b],
                out_hbm.at[pl.ds(pl.multiple_of(wbase + c * C, C), C)])

        load_idx(0, 0)
        fire(0)

        def body(i, carry):
            c0 = 2 * i
            load_idx(c0 + 1, 1)
            fire(1)
            wait(0)
            store(c0, 0)

            @pl.when(i < nc2 - 1)
            def _():
                load_idx(c0 + 2, 0)
                fire(0)

            wait(1)
            store(c0 + 1, 1)
            return carry

        lax.fori_loop(0, nc2, body, 0)

    return k


def _sc_gather(table, idx_flat, D):
    B0 = idx_flat.shape[0]
    B = _pad_len(B0, D)
    if B != B0:
        idx_flat = jnp.concatenate(
            [idx_flat, jnp.zeros((B - B0,), jnp.int32)])
    T = table.shape[0]
    staged = T * D <= 900_000
    return _sc_gather_fn(B, D, T, staged)(table, idx_flat)


BLK = 2000
NBLK = N // BLK


@functools.lru_cache(None)
def _conv_fn(B):
    def body(g_ref, w_ref, b_ref, o_ref):
        i = pl.program_id(1)

        @pl.when(i == 0)
        def _():
            o_ref[...] = jnp.zeros_like(o_ref)

        o_ref[...] += jnp.dot(g_ref[...], w_ref[0],
                              preferred_element_type=jnp.float32)

        @pl.when(i == 26)
        def _():
            o_ref[...] = jnp.maximum(o_ref[...] + b_ref[...], 0.0)

    return pl.pallas_call(
        body,
        grid=(NBLK, 27),
        in_specs=[
            pl.BlockSpec((BLK, CH), lambda j, i: (i * NBLK + j, 0)),
            pl.BlockSpec((1, CH, CH), lambda j, i: (i, 0, 0)),
            pl.BlockSpec((1, CH), lambda j, i: (0, 0)),
        ],
        out_specs=pl.BlockSpec((BLK, CH), lambda j, i: (j, 0)),
        out_shape=jax.ShapeDtypeStruct((N, CH), jnp.float32),
    )


@functools.lru_cache(None)
def _conv2_fn(B):
    def body(g1_ref, g2_ref, w1_ref, w2_ref, b_ref, o_ref):
        i = pl.program_id(1)

        @pl.when(i == 0)
        def _():
            o_ref[...] = jnp.zeros_like(o_ref)

        o_ref[...] += (
            jnp.dot(g1_ref[...], w1_ref[0],
                    preferred_element_type=jnp.float32)
            + jnp.dot(g2_ref[...], w2_ref[0],
                      preferred_element_type=jnp.float32))

        @pl.when(i == 26)
        def _():
            o_ref[...] = jnp.maximum(o_ref[...] + b_ref[...], 0.0)

    gspec = pl.BlockSpec((BLK, CH), lambda j, i: (i * NBLK + j, 0))
    wspec = pl.BlockSpec((1, CH, CH), lambda j, i: (i, 0, 0))
    return pl.pallas_call(
        body,
        grid=(NBLK, 27),
        in_specs=[gspec, gspec, wspec, wspec,
                  pl.BlockSpec((1, CH), lambda j, i: (0, 0))],
        out_specs=pl.BlockSpec((BLK, CH), lambda j, i: (j, 0)),
        out_shape=jax.ShapeDtypeStruct((N, CH), jnp.float32),
    )


@functools.lru_cache(None)
def _poolmax_fn(B):
    def body(g_ref, o_ref):
        t = pl.program_id(1)

        @pl.when(t == 0)
        def _():
            o_ref[...] = g_ref[...]

        @pl.when(t > 0)
        def _():
            o_ref[...] = jnp.maximum(o_ref[...], g_ref[...])

    return pl.pallas_call(
        body,
        grid=(NBLK, 8),
        in_specs=[pl.BlockSpec((BLK, CH), lambda j, t: (t * NBLK + j, 0))],
        out_specs=pl.BlockSpec((BLK, CH), lambda j, t: (j, 0)),
        out_shape=jax.ShapeDtypeStruct((N, CH), jnp.float32),
    )


@functools.lru_cache(None)
def _head_fn():
    def body(x_ref, w_ref, b_ref, o_ref):
        o_ref[...] = jnp.dot(x_ref[...], w_ref[...],
                             preferred_element_type=jnp.float32) + b_ref[...]

    return pl.pallas_call(
        body,
        grid=(NBLK,),
        in_specs=[
            pl.BlockSpec((BLK, CH), lambda j: (j, 0)),
            pl.BlockSpec((CH, NCLASS), lambda j: (0, 0)),
            pl.BlockSpec((1, NCLASS), lambda j: (0, 0)),
        ],
        out_specs=pl.BlockSpec((BLK, NCLASS), lambda j: (j, 0)),
        out_shape=jax.ShapeDtypeStruct((N, NCLASS), jnp.float32),
    )


def _lin(c, g):
    return (c[..., 0] * g + c[..., 1]) * g + c[..., 2]


def _neighbor_idx(coords, keys, valid, g):
    g3 = g * g * g
    R = g3 // 8 + 2
    ar = jnp.arange(N, dtype=jnp.int32)
    wk = jnp.where(valid, keys, jnp.int32(-10))
    tbl = jnp.full((R, 16), np.int32(-1))
    tbl = tbl.at[wk // 8, wk % 8 + 1].set(ar, mode="drop")
    tbl = tbl.at[jnp.where(wk % 8 == 7, wk // 8 + 1, R), 0].set(
        ar, mode="drop")
    tbl = tbl.at[jnp.where(wk % 8 == 0, wk // 8 - 1, R), 9].set(
        ar, mode="drop")
    d9 = np.array([dx * g * g + dy * g for dx in (-1, 0, 1)
                   for dy in (-1, 0, 1)], np.int32)
    q = jnp.clip(keys[None, :] + d9[:, None], 0, g3 - 1)
    gath = _sc_gather(lax.bitcast_convert_type(tbl, jnp.float32),
                      (q // 8).reshape(-1), 16)
    rows9 = lax.bitcast_convert_type(
        gath[:9 * N].reshape(9, N, 16), jnp.int32)
    cbase = q % 8 + 1
    cols = jnp.arange(16, dtype=jnp.int32)
    pos = []
    for dz in (-1, 0, 1):
        sel = cols[None, None, :] == (cbase + dz)[:, :, None]
        pos.append(jnp.sum(jnp.where(sel, rows9, 0), axis=-1))
    pos27 = jnp.stack(pos, axis=1).reshape(27, N)
    nc = coords.T[None] + OFFS[:, :, None]
    inb = jnp.all((nc >= 0) & (nc < g), axis=1)
    gidx = jnp.where((pos27 >= 0) & inb, pos27, N).astype(jnp.int32)
    return gidx.reshape(-1)


def _pool_build(coords, keys, valid, g):
    gc = g // 2
    gc3 = gc * gc * gc
    ar = jnp.arange(N, dtype=jnp.int32)
    ck = jnp.where(valid, _lin(coords // 2, gc), gc3).astype(jnp.int32)
    occ = jnp.zeros((gc3,), jnp.int32).at[ck].set(1, mode="drop")
    csum = jnp.cumsum(occ)
    M = csum[-1]
    slot = (csum - occ).astype(jnp.int32)
    cells = jnp.zeros((N,), jnp.int32).at[
        jnp.where(occ == 1, slot, N)].set(
            jnp.arange(gc3, dtype=jnp.int32), mode="drop")
    pos_u = jnp.minimum(slot[jnp.clip(ck, 0, gc3 - 1)], N - 1)
    sub = (((coords[:, 0] & 1) << 2) | ((coords[:, 1] & 1) << 1)
           | (coords[:, 2] & 1))
    sub = jnp.where(valid, sub, 8)
    cidx = jnp.full((8, N), N, jnp.int32).at[sub, pos_u].set(ar, mode="drop")
    new_valid = ar < M
    new_keys = jnp.where(new_valid, cells, SENT)
    uk = jnp.where(new_valid, cells, 0)
    new_coords = jnp.stack(
        [uk // (gc * gc), (uk // gc) % gc, uk % gc], axis=1).astype(jnp.int32)
    return cidx.reshape(-1), pos_u, new_coords, new_keys, new_valid


def _ext(feat):
    return jnp.concatenate([feat, jnp.zeros((1, feat.shape[1]), feat.dtype)])


def _conv(feat, gidx, W, b):
    g = _sc_gather(_ext(feat), gidx, CH)
    return _conv_fn(g.shape[0])(g, W, b.reshape(1, CH))


def _conv2(up, skip, gidx, W, b):
    g1 = _sc_gather(_ext(up), gidx, CH)
    g2 = _sc_gather(_ext(skip), gidx, CH)
    return _conv2_fn(g1.shape[0])(
        g1, g2, W[:, :CH, :], W[:, CH:, :], b.reshape(1, CH))


def kernel(voxel_features, voxel_xyz_indices, num_valid_voxels,
           W_b0, W_b1, W_d00, W_d01, W_d10, W_d11, W_d20, W_d21,
           W_e00, W_e01, W_e10, W_e11, W_e20, W_e21, W_h,
           b_b0, b_b1, b_d00, b_d01, b_d10, b_d11, b_d20, b_d21,
           b_e00, b_e01, b_e10, b_e11, b_e20, b_e21, b_h):
    P = dict(W_b0=W_b0, W_b1=W_b1, W_d00=W_d00, W_d01=W_d01, W_d10=W_d10,
             W_d11=W_d11, W_d20=W_d20, W_d21=W_d21, W_e00=W_e00,
             W_e01=W_e01, W_e10=W_e10, W_e11=W_e11, W_e20=W_e20,
             W_e21=W_e21, b_b0=b_b0, b_b1=b_b1, b_d00=b_d00, b_d01=b_d01,
             b_d10=b_d10, b_d11=b_d11, b_d20=b_d20, b_d21=b_d21,
             b_e00=b_e00, b_e01=b_e01, b_e10=b_e10, b_e11=b_e11,
             b_e20=b_e20, b_e21=b_e21)
    feat = voxel_features[0]
    coords = voxel_xyz_indices[0].astype(jnp.int32)

    g = G
    keys = _lin(coords, g).astype(jnp.int32)
    valid = jnp.ones((N,), bool)

    skips = []
    for lvl, blk in enumerate((('e00', 'e01'), ('e10', 'e11'),
                               ('e20', 'e21'))):
        gidx = _neighbor_idx(coords, keys, valid, g)
        for nm in blk:
            feat = _conv(feat, gidx, P['W_' + nm], P['b_' + nm])
        cidx, pos_u, nco, nke, nva = _pool_build(coords, keys, valid, g)
        skips.append((feat, gidx, pos_u))
        gch = _sc_gather(_ext(feat), cidx, CH)
        feat = _poolmax_fn(gch.shape[0])(gch)
        coords, keys, valid, g = nco, nke, nva, g // 2

    gidx = _neighbor_idx(coords, keys, valid, g)
    for nm in ('b0', 'b1'):
        feat = _conv(feat, gidx, P['W_' + nm], P['b_' + nm])

    for blk, (sf, sgidx, pos_u) in zip(
            (('d00', 'd01'), ('d10', 'd11'), ('d20', 'd21')),
            reversed(skips)):
        up = _sc_gather(feat, pos_u, CH)[:N]
        feat = _conv2(up, sf, sgidx, P['W_' + blk[0]], P['b_' + blk[0]])
        feat = _conv(feat, sgidx, P['W_' + blk[1]], P['b_' + blk[1]])

    return _head_fn()(feat, W_h, b_h.reshape(1, NCLASS))
